# Initial kernel scaffold; baseline (speedup 1.0000x reference)
import jax
import jax.numpy as jnp
from jax.experimental import pallas as pl


def kernel(node_feat, edge_index, W1, b1, gamma, beta, W2, b2):
    raise NotImplementedError("write your pallas kernel here")


# speedup vs baseline: 13.5198x; 13.5198x over previous
import functools

import jax
import jax.numpy as jnp
from jax import lax
from jax.experimental import pallas as pl
from jax.experimental.pallas import tpu as pltpu
from jax.experimental.pallas import tpu_sc as plsc

N = 10000
E = 320000
D_IN = 128
D_H = 256
D_OUT = 128
EPS = 1e-5

NC = 2
NS = 16
NW = NC * NS
EPT = E // NW
CHUNK = 80
NCHUNK = EPT // CHUNK
RPT = 624
RPT_LAST = N - (NS - 1) * RPT
_MESH = plsc.VectorSubcoreMesh(core_axis_name="c", subcore_axis_name="s")


def _per_tile_rows(s, copy_fn):
    @pl.when(s < NS - 1)
    def _():
        copy_fn(pl.multiple_of(s * RPT, 8), RPT)

    @pl.when(s == NS - 1)
    def _():
        copy_fn((NS - 1) * RPT, RPT_LAST)


DW = 128


def _deg_body(dst_hbm, zeros_hbm, ones_hbm, out_hbm, shared, ones_v, idx_d):
    c = lax.axis_index("c")
    s = lax.axis_index("s")
    wid = c * NS + s

    _per_tile_rows(s, lambda r0, nr: pltpu.sync_copy(
        zeros_hbm.at[pl.ds(r0, nr)], shared.at[pl.ds(r0, nr)]))
    pltpu.sync_copy(ones_hbm, ones_v)
    plsc.subcore_barrier()

    base = wid * EPT

    def body(i, carry):
        off = pl.multiple_of(base + i * CHUNK, 8)
        pltpu.sync_copy(dst_hbm.at[pl.ds(off, CHUNK)], idx_d)
        pltpu.sync_copy(ones_v, shared.at[idx_d], add=True)
        return carry

    lax.fori_loop(0, NCHUNK, body, 0)
    plsc.subcore_barrier()
    _per_tile_rows(s, lambda r0, nr: pltpu.sync_copy(
        shared.at[pl.ds(r0, nr)], out_hbm.at[c, pl.ds(r0, nr)]))


_deg_kernel = functools.partial(
    pl.kernel,
    out_type=jax.ShapeDtypeStruct((NC, N, DW), jnp.float32),
    mesh=_MESH,
    scratch_types=[
        pltpu.VMEM_SHARED((N, DW), jnp.float32),
        pltpu.VMEM((CHUNK, DW), jnp.float32),
        pltpu.VMEM((CHUNK,), jnp.int32),
    ],
)(_deg_body)


def _agg_body(u_hbm, src_hbm, dst_hbm, zeros_hbm, out_hbm,
              shared, idx_s, idx_d, rows, sem):
    c = lax.axis_index("c")
    s = lax.axis_index("s")
    wid = c * NS + s

    @pl.when(c == 0)
    def _():
        _per_tile_rows(s, lambda r0, nr: pltpu.sync_copy(
            u_hbm.at[pl.ds(r0, nr)], shared.at[pl.ds(r0, nr)]))

    @pl.when(c != 0)
    def _():
        _per_tile_rows(s, lambda r0, nr: pltpu.sync_copy(
            zeros_hbm.at[pl.ds(r0, nr)], shared.at[pl.ds(r0, nr)]))

    plsc.subcore_barrier()

    base = wid * EPT

    def body(i, carry):
        off = pl.multiple_of(base + i * CHUNK, 8)
        pltpu.sync_copy(src_hbm.at[pl.ds(off, CHUNK)], idx_s)
        pltpu.sync_copy(dst_hbm.at[pl.ds(off, CHUNK)], idx_d)
        pltpu.async_copy(u_hbm.at[idx_s], rows, sem).wait()
        pltpu.sync_copy(rows, shared.at[idx_d], add=True)
        return carry

    lax.fori_loop(0, NCHUNK, body, 0)
    plsc.subcore_barrier()
    _per_tile_rows(s, lambda r0, nr: pltpu.sync_copy(
        shared.at[pl.ds(r0, nr)], out_hbm.at[c, pl.ds(r0, nr)]))


_agg_kernel = functools.partial(
    pl.kernel,
    out_type=jax.ShapeDtypeStruct((NC, N, D_IN), jnp.float32),
    mesh=_MESH,
    scratch_types=[
        pltpu.VMEM_SHARED((N, D_IN), jnp.float32),
        pltpu.VMEM((CHUNK,), jnp.int32),
        pltpu.VMEM((CHUNK,), jnp.int32),
        pltpu.VMEM((CHUNK, D_IN), jnp.float32),
        pltpu.SemaphoreType.DMA,
    ],
)(_agg_body)


BM = 1000
GRID = N // BM


def _dinv_of(p):
    return lax.rsqrt(1.0 + jnp.sum(p, axis=(0, 2)) * (1.0 / DW))[:, None]


def _tc_scale_body(degp_ref, x_ref, u_ref):
    u_ref[...] = x_ref[...] * _dinv_of(degp_ref[...])


def _tc_l1_body(degp_ref, vp_ref, w1_ref, h_ref, stats_ref):
    i = pl.program_id(0)
    dinv = _dinv_of(degp_ref[...])
    vp = vp_ref[...]
    w = (vp[0] + vp[1]) * dinv
    h = jnp.dot(w, w1_ref[...], preferred_element_type=jnp.float32)
    h_ref[...] = h
    st = jnp.concatenate(
        [jnp.sum(h, axis=0, keepdims=True), jnp.sum(h * h, axis=0, keepdims=True)],
        axis=0,
    )

    @pl.when(i == 0)
    def _():
        stats_ref[...] = st

    @pl.when(i != 0)
    def _():
        stats_ref[...] += st


def _tc_l2_body(degp_ref, h_ref, stats_ref, gamma_ref, beta_ref, w2_ref, u2_ref):
    dinv = _dinv_of(degp_ref[...])
    st = stats_ref[...]
    mean = st[0:1] * (1.0 / N)
    var = st[1:2] * (1.0 / N) - mean * mean
    scale = gamma_ref[...] * lax.rsqrt(var + EPS)
    x2 = jnp.maximum((h_ref[...] - mean) * scale + beta_ref[...], 0.0)
    u2_ref[...] = jnp.dot(x2, w2_ref[...], preferred_element_type=jnp.float32) * dinv


def _tc_out_body(degp_ref, vp_ref, b2_ref, out_ref):
    dinv = _dinv_of(degp_ref[...])
    vp = vp_ref[...]
    out_ref[...] = (vp[0] + vp[1]) * dinv + b2_ref[...]


_degp_spec = pl.BlockSpec((NC, BM, DW), lambda i: (0, i, 0))
_vp_spec = pl.BlockSpec((NC, BM, D_IN), lambda i: (0, i, 0))
_row128_spec = pl.BlockSpec((BM, D_IN), lambda i: (i, 0))
_row256_spec = pl.BlockSpec((BM, D_H), lambda i: (i, 0))
_full = lambda shape: pl.BlockSpec(shape, lambda i: tuple(0 for _ in shape))

_tc_scale = pl.pallas_call(
    _tc_scale_body,
    grid=(GRID,),
    in_specs=[_degp_spec, _row128_spec],
    out_specs=_row128_spec,
    out_shape=jax.ShapeDtypeStruct((N, D_IN), jnp.float32),
)

_tc_l1 = pl.pallas_call(
    _tc_l1_body,
    grid=(GRID,),
    in_specs=[_degp_spec, _vp_spec, _full((D_IN, D_H))],
    out_specs=[_row256_spec, _full((2, D_H))],
    out_shape=[
        jax.ShapeDtypeStruct((N, D_H), jnp.float32),
        jax.ShapeDtypeStruct((2, D_H), jnp.float32),
    ],
)

_tc_l2 = pl.pallas_call(
    _tc_l2_body,
    grid=(GRID,),
    in_specs=[_degp_spec, _row256_spec, _full((2, D_H)), _full((1, D_H)),
              _full((1, D_H)), _full((D_H, D_OUT))],
    out_specs=_row128_spec,
    out_shape=jax.ShapeDtypeStruct((N, D_OUT), jnp.float32),
)

_tc_out = pl.pallas_call(
    _tc_out_body,
    grid=(GRID,),
    in_specs=[_degp_spec, _vp_spec, _full((1, D_OUT))],
    out_specs=_row128_spec,
    out_shape=jax.ShapeDtypeStruct((N, D_OUT), jnp.float32),
)


def kernel(node_feat, edge_index, W1, b1, gamma, beta, W2, b2):
    del b1
    src = edge_index[0]
    dst = edge_index[1]
    zeros128 = jnp.zeros((N, D_IN), jnp.float32)

    degp = _deg_kernel(dst, zeros128, jnp.ones((CHUNK, DW), jnp.float32))

    u1 = _tc_scale(degp, node_feat)
    v1p = _agg_kernel(u1, src, dst, zeros128)
    h, stats = _tc_l1(degp, v1p, W1)
    u2 = _tc_l2(degp, h, stats, gamma.reshape(1, D_H), beta.reshape(1, D_H), W2)
    v2p = _agg_kernel(u2, src, dst, zeros128)
    return _tc_out(degp, v2p, b2.reshape(1, D_OUT))


# speedup vs baseline: 29.6603x; 2.1938x over previous
import functools

import jax
import jax.numpy as jnp
from jax import lax
from jax.experimental import pallas as pl
from jax.experimental.pallas import tpu as pltpu
from jax.experimental.pallas import tpu_sc as plsc

N = 10000
E = 320000
D_IN = 128
D_H = 256
D_OUT = 128
EPS = 1e-5

NC = 2
NS = 16
NW = NC * NS
EPT = E // NW
CHUNK = 125
NCHUNK = EPT // CHUNK
NPAIR = NCHUNK // 2
HCH = NCHUNK // 2
RPT = 624
RPT_LAST = N - (NS - 1) * RPT
_MESH = plsc.VectorSubcoreMesh(core_axis_name="c", subcore_axis_name="s")


def _per_tile_rows(s, copy_fn):
    @pl.when(s < NS - 1)
    def _():
        copy_fn(pl.multiple_of(s * RPT, 8), RPT)

    @pl.when(s == NS - 1)
    def _():
        copy_fn((NS - 1) * RPT, RPT_LAST)


DW = 128


def _deg_body(dst2_hbm, zeros_hbm, ones_hbm, out_hbm, shared, ones_v, idx_d2):
    c = lax.axis_index("c")
    s = lax.axis_index("s")
    wid = c * NS + s

    _per_tile_rows(s, lambda r0, nr: pltpu.sync_copy(
        zeros_hbm.at[pl.ds(r0, nr)], shared.at[pl.ds(r0, nr)]))
    pltpu.sync_copy(ones_hbm, ones_v)
    pltpu.sync_copy(dst2_hbm.at[wid], idx_d2)
    plsc.subcore_barrier()

    def body(j, carry):
        pltpu.sync_copy(ones_v, shared.at[idx_d2.at[j]], add=True)
        return carry

    lax.fori_loop(0, NCHUNK, body, 0)
    plsc.subcore_barrier()
    _per_tile_rows(s, lambda r0, nr: pltpu.sync_copy(
        shared.at[pl.ds(r0, nr)], out_hbm.at[c, pl.ds(r0, nr)]))


_deg_kernel = functools.partial(
    pl.kernel,
    out_type=jax.ShapeDtypeStruct((NC, N, DW), jnp.float32),
    mesh=_MESH,
    scratch_types=[
        pltpu.VMEM_SHARED((N, DW), jnp.float32),
        pltpu.VMEM((CHUNK, DW), jnp.float32),
        pltpu.VMEM((NCHUNK, CHUNK), jnp.int32),
    ],
)(_deg_body)


def _agg_body(u_hbm, src2_hbm, dst2_hbm, zeros_hbm, out_hbm,
              shared, idx_s2, idx_d2, rows0, rows1, gsem0, gsem1):
    c = lax.axis_index("c")
    s = lax.axis_index("s")
    wid = c * NS + s

    @pl.when(c == 0)
    def _():
        _per_tile_rows(s, lambda r0, nr: pltpu.sync_copy(
            u_hbm.at[pl.ds(r0, nr)], shared.at[pl.ds(r0, nr)]))

    @pl.when(c != 0)
    def _():
        _per_tile_rows(s, lambda r0, nr: pltpu.sync_copy(
            zeros_hbm.at[pl.ds(r0, nr)], shared.at[pl.ds(r0, nr)]))

    plsc.subcore_barrier()

    for h in range(2):
        pltpu.sync_copy(src2_hbm.at[wid, pl.ds(h * HCH, HCH)], idx_s2)
        pltpu.sync_copy(dst2_hbm.at[wid, pl.ds(h * HCH, HCH)], idx_d2)
        pltpu.async_copy(u_hbm.at[idx_s2.at[0]], rows0, gsem0)
        pltpu.async_copy(u_hbm.at[idx_s2.at[1]], rows1, gsem1)

        def pair(k, carry):
            j0 = k * 2
            j1 = j0 + 1
            pltpu.make_async_copy(u_hbm.at[idx_s2.at[j0]], rows0, gsem0).wait()
            pltpu.sync_copy(rows0, shared.at[idx_d2.at[j0]], add=True)

            @pl.when(j0 + 2 < HCH)
            def _():
                pltpu.async_copy(u_hbm.at[idx_s2.at[j0 + 2]], rows0, gsem0)

            pltpu.make_async_copy(u_hbm.at[idx_s2.at[j1]], rows1, gsem1).wait()
            pltpu.sync_copy(rows1, shared.at[idx_d2.at[j1]], add=True)

            @pl.when(j1 + 2 < HCH)
            def _():
                pltpu.async_copy(u_hbm.at[idx_s2.at[j1 + 2]], rows1, gsem1)

            return carry

        lax.fori_loop(0, HCH // 2, pair, 0)

    plsc.subcore_barrier()
    _per_tile_rows(s, lambda r0, nr: pltpu.sync_copy(
        shared.at[pl.ds(r0, nr)], out_hbm.at[c, pl.ds(r0, nr)]))


_agg_kernel = functools.partial(
    pl.kernel,
    out_type=jax.ShapeDtypeStruct((NC, N, D_IN), jnp.float32),
    mesh=_MESH,
    scratch_types=[
        pltpu.VMEM_SHARED((N, D_IN), jnp.float32),
        pltpu.VMEM((HCH, CHUNK), jnp.int32),
        pltpu.VMEM((HCH, CHUNK), jnp.int32),
        pltpu.VMEM((CHUNK, D_IN), jnp.float32),
        pltpu.VMEM((CHUNK, D_IN), jnp.float32),
        pltpu.SemaphoreType.DMA,
        pltpu.SemaphoreType.DMA,
    ],
)(_agg_body)


BM = 1000
GRID = N // BM


def _dinv_of(p):
    return lax.rsqrt(1.0 + jnp.sum(p, axis=(0, 2)) * (1.0 / DW))[:, None]


def _tc_scale_body(degp_ref, x_ref, u_ref):
    u_ref[...] = x_ref[...] * _dinv_of(degp_ref[...])


def _tc_l1_body(degp_ref, vp_ref, w1_ref, h_ref, stats_ref):
    i = pl.program_id(0)
    dinv = _dinv_of(degp_ref[...])
    vp = vp_ref[...]
    w = (vp[0] + vp[1]) * dinv
    h = jnp.dot(w, w1_ref[...], preferred_element_type=jnp.float32)
    h_ref[...] = h
    st = jnp.concatenate(
        [jnp.sum(h, axis=0, keepdims=True), jnp.sum(h * h, axis=0, keepdims=True)],
        axis=0,
    )

    @pl.when(i == 0)
    def _():
        stats_ref[...] = st

    @pl.when(i != 0)
    def _():
        stats_ref[...] += st


def _tc_l2_body(degp_ref, h_ref, stats_ref, gamma_ref, beta_ref, w2_ref, u2_ref):
    dinv = _dinv_of(degp_ref[...])
    st = stats_ref[...]
    mean = st[0:1] * (1.0 / N)
    var = st[1:2] * (1.0 / N) - mean * mean
    scale = gamma_ref[...] * lax.rsqrt(var + EPS)
    x2 = jnp.maximum((h_ref[...] - mean) * scale + beta_ref[...], 0.0)
    u2_ref[...] = jnp.dot(x2, w2_ref[...], preferred_element_type=jnp.float32) * dinv


def _tc_out_body(degp_ref, vp_ref, b2_ref, out_ref):
    dinv = _dinv_of(degp_ref[...])
    vp = vp_ref[...]
    out_ref[...] = (vp[0] + vp[1]) * dinv + b2_ref[...]


_degp_spec = pl.BlockSpec((NC, BM, DW), lambda i: (0, i, 0))
_vp_spec = pl.BlockSpec((NC, BM, D_IN), lambda i: (0, i, 0))
_row128_spec = pl.BlockSpec((BM, D_IN), lambda i: (i, 0))
_row256_spec = pl.BlockSpec((BM, D_H), lambda i: (i, 0))
_full = lambda shape: pl.BlockSpec(shape, lambda i: tuple(0 for _ in shape))

_tc_scale = pl.pallas_call(
    _tc_scale_body,
    grid=(GRID,),
    in_specs=[_degp_spec, _row128_spec],
    out_specs=_row128_spec,
    out_shape=jax.ShapeDtypeStruct((N, D_IN), jnp.float32),
)

_tc_l1 = pl.pallas_call(
    _tc_l1_body,
    grid=(GRID,),
    in_specs=[_degp_spec, _vp_spec, _full((D_IN, D_H))],
    out_specs=[_row256_spec, _full((2, D_H))],
    out_shape=[
        jax.ShapeDtypeStruct((N, D_H), jnp.float32),
        jax.ShapeDtypeStruct((2, D_H), jnp.float32),
    ],
)

_tc_l2 = pl.pallas_call(
    _tc_l2_body,
    grid=(GRID,),
    in_specs=[_degp_spec, _row256_spec, _full((2, D_H)), _full((1, D_H)),
              _full((1, D_H)), _full((D_H, D_OUT))],
    out_specs=_row128_spec,
    out_shape=jax.ShapeDtypeStruct((N, D_OUT), jnp.float32),
)

_tc_out = pl.pallas_call(
    _tc_out_body,
    grid=(GRID,),
    in_specs=[_degp_spec, _vp_spec, _full((1, D_OUT))],
    out_specs=_row128_spec,
    out_shape=jax.ShapeDtypeStruct((N, D_OUT), jnp.float32),
)


def kernel(node_feat, edge_index, W1, b1, gamma, beta, W2, b2):
    del b1
    src2 = edge_index[0].reshape(NW, NCHUNK, CHUNK)
    dst2 = edge_index[1].reshape(NW, NCHUNK, CHUNK)
    zeros128 = jnp.zeros((N, D_IN), jnp.float32)

    degp = _deg_kernel(dst2, zeros128, jnp.ones((CHUNK, DW), jnp.float32))

    u1 = _tc_scale(degp, node_feat)
    v1p = _agg_kernel(u1, src2, dst2, zeros128)
    h, stats = _tc_l1(degp, v1p, W1)
    u2 = _tc_l2(degp, h, stats, gamma.reshape(1, D_H), beta.reshape(1, D_H), W2)
    v2p = _agg_kernel(u2, src2, dst2, zeros128)
    return _tc_out(degp, v2p, b2.reshape(1, D_OUT))


# speedup vs baseline: 29.8251x; 1.0056x over previous
import functools

import jax
import jax.numpy as jnp
from jax import lax
from jax.experimental import pallas as pl
from jax.experimental.pallas import tpu as pltpu
from jax.experimental.pallas import tpu_sc as plsc

N = 10000
E = 320000
D_IN = 128
D_H = 256
D_OUT = 128
EPS = 1e-5

NC = 2
NS = 16
NW = NC * NS
EPT = E // NW
CHUNK = 125
NCHUNK = EPT // CHUNK
NPAIR = NCHUNK // 2
HCH = NCHUNK // 2
RPT = 624
RPT_LAST = N - (NS - 1) * RPT
_MESH = plsc.VectorSubcoreMesh(core_axis_name="c", subcore_axis_name="s")


def _per_tile_rows(s, copy_fn):
    @pl.when(s < NS - 1)
    def _():
        copy_fn(pl.multiple_of(s * RPT, 8), RPT)

    @pl.when(s == NS - 1)
    def _():
        copy_fn((NS - 1) * RPT, RPT_LAST)


DW = 128


def _deg_body(dst2_hbm, zeros_hbm, ones_hbm, out_hbm, shared, ones_v, idx_d2):
    c = lax.axis_index("c")
    s = lax.axis_index("s")
    wid = c * NS + s

    _per_tile_rows(s, lambda r0, nr: pltpu.sync_copy(
        zeros_hbm.at[pl.ds(r0, nr)], shared.at[pl.ds(r0, nr)]))
    pltpu.sync_copy(ones_hbm, ones_v)
    pltpu.sync_copy(dst2_hbm.at[wid], idx_d2)
    plsc.subcore_barrier()

    def body(j, carry):
        pltpu.sync_copy(ones_v, shared.at[idx_d2.at[j]], add=True)
        return carry

    lax.fori_loop(0, NCHUNK, body, 0)
    plsc.subcore_barrier()
    _per_tile_rows(s, lambda r0, nr: pltpu.sync_copy(
        shared.at[pl.ds(r0, nr)], out_hbm.at[c, pl.ds(r0, nr)]))


_deg_kernel = functools.partial(
    pl.kernel,
    out_type=jax.ShapeDtypeStruct((NC, N, DW), jnp.float32),
    mesh=_MESH,
    scratch_types=[
        pltpu.VMEM_SHARED((N, DW), jnp.float32),
        pltpu.VMEM((CHUNK, DW), jnp.float32),
        pltpu.VMEM((NCHUNK, CHUNK), jnp.int32),
    ],
)(_deg_body)


def _agg_body(u_hbm, src2_hbm, dst2_hbm, zeros_hbm, out_hbm,
              shared, idx_s2, idx_d2, rows0, rows1, gsem0, gsem1):
    c = lax.axis_index("c")
    s = lax.axis_index("s")
    wid = c * NS + s

    @pl.when(c == 0)
    def _():
        _per_tile_rows(s, lambda r0, nr: pltpu.sync_copy(
            u_hbm.at[pl.ds(r0, nr)], shared.at[pl.ds(r0, nr)]))

    @pl.when(c != 0)
    def _():
        _per_tile_rows(s, lambda r0, nr: pltpu.sync_copy(
            zeros_hbm.at[pl.ds(r0, nr)], shared.at[pl.ds(r0, nr)]))

    plsc.subcore_barrier()

    for h in range(2):
        pltpu.sync_copy(src2_hbm.at[wid, pl.ds(h * HCH, HCH)], idx_s2)
        pltpu.sync_copy(dst2_hbm.at[wid, pl.ds(h * HCH, HCH)], idx_d2)
        pltpu.async_copy(u_hbm.at[idx_s2.at[0]], rows0, gsem0)
        pltpu.async_copy(u_hbm.at[idx_s2.at[1]], rows1, gsem1)

        def pair(k, carry):
            j0 = k * 2
            j1 = j0 + 1
            pltpu.make_async_copy(u_hbm.at[idx_s2.at[j0]], rows0, gsem0).wait()
            pltpu.sync_copy(rows0, shared.at[idx_d2.at[j0]], add=True)

            @pl.when(j0 + 2 < HCH)
            def _():
                pltpu.async_copy(u_hbm.at[idx_s2.at[j0 + 2]], rows0, gsem0)

            pltpu.make_async_copy(u_hbm.at[idx_s2.at[j1]], rows1, gsem1).wait()
            pltpu.sync_copy(rows1, shared.at[idx_d2.at[j1]], add=True)

            @pl.when(j1 + 2 < HCH)
            def _():
                pltpu.async_copy(u_hbm.at[idx_s2.at[j1 + 2]], rows1, gsem1)

            return carry

        lax.fori_loop(0, HCH // 2, pair, 0)

    plsc.subcore_barrier()
    _per_tile_rows(s, lambda r0, nr: pltpu.sync_copy(
        shared.at[pl.ds(r0, nr)], out_hbm.at[c, pl.ds(r0, nr)]))


_agg_kernel = functools.partial(
    pl.kernel,
    out_type=jax.ShapeDtypeStruct((NC, N, D_IN), jnp.float32),
    mesh=_MESH,
    scratch_types=[
        pltpu.VMEM_SHARED((N, D_IN), jnp.float32),
        pltpu.VMEM((HCH, CHUNK), jnp.int32),
        pltpu.VMEM((HCH, CHUNK), jnp.int32),
        pltpu.VMEM((CHUNK, D_IN), jnp.float32),
        pltpu.VMEM((CHUNK, D_IN), jnp.float32),
        pltpu.SemaphoreType.DMA,
        pltpu.SemaphoreType.DMA,
    ],
)(_agg_body)


BM = 1000
GRID = N // BM


def _dinv_of(p):
    return lax.rsqrt(1.0 + jnp.sum(p, axis=(0, 2)) * (1.0 / DW))[:, None]


def _tc_scale_body(degp_ref, x_ref, u_ref, dinv_ref):
    dinv = _dinv_of(degp_ref[...])
    u_ref[...] = x_ref[...] * dinv
    dinv_ref[...] = dinv


def _tc_l1_body(dinv_ref, vp_ref, w1_ref, h_ref, stats_ref):
    i = pl.program_id(0)
    dinv = dinv_ref[...]
    vp = vp_ref[...]
    w = (vp[0] + vp[1]) * dinv
    h = jnp.dot(w, w1_ref[...], preferred_element_type=jnp.float32)
    h_ref[...] = h
    st = jnp.concatenate(
        [jnp.sum(h, axis=0, keepdims=True), jnp.sum(h * h, axis=0, keepdims=True)],
        axis=0,
    )

    @pl.when(i == 0)
    def _():
        stats_ref[...] = st

    @pl.when(i != 0)
    def _():
        stats_ref[...] += st


def _tc_l2_body(dinv_ref, h_ref, stats_ref, gamma_ref, beta_ref, w2_ref, u2_ref):
    dinv = dinv_ref[...]
    st = stats_ref[...]
    mean = st[0:1] * (1.0 / N)
    var = st[1:2] * (1.0 / N) - mean * mean
    scale = gamma_ref[...] * lax.rsqrt(var + EPS)
    x2 = jnp.maximum((h_ref[...] - mean) * scale + beta_ref[...], 0.0)
    u2_ref[...] = jnp.dot(x2, w2_ref[...], preferred_element_type=jnp.float32) * dinv


def _tc_out_body(dinv_ref, vp_ref, b2_ref, out_ref):
    dinv = dinv_ref[...]
    vp = vp_ref[...]
    out_ref[...] = (vp[0] + vp[1]) * dinv + b2_ref[...]


_degp_spec = pl.BlockSpec((NC, BM, DW), lambda i: (0, i, 0))
_dinv_spec = pl.BlockSpec((BM, 1), lambda i: (i, 0))
_vp_spec = pl.BlockSpec((NC, BM, D_IN), lambda i: (0, i, 0))
_row128_spec = pl.BlockSpec((BM, D_IN), lambda i: (i, 0))
_row256_spec = pl.BlockSpec((BM, D_H), lambda i: (i, 0))
_full = lambda shape: pl.BlockSpec(shape, lambda i: tuple(0 for _ in shape))

_tc_scale = pl.pallas_call(
    _tc_scale_body,
    grid=(GRID,),
    in_specs=[_degp_spec, _row128_spec],
    out_specs=[_row128_spec, _dinv_spec],
    out_shape=[
        jax.ShapeDtypeStruct((N, D_IN), jnp.float32),
        jax.ShapeDtypeStruct((N, 1), jnp.float32),
    ],
)

_tc_l1 = pl.pallas_call(
    _tc_l1_body,
    grid=(GRID,),
    in_specs=[_dinv_spec, _vp_spec, _full((D_IN, D_H))],
    out_specs=[_row256_spec, _full((2, D_H))],
    out_shape=[
        jax.ShapeDtypeStruct((N, D_H), jnp.float32),
        jax.ShapeDtypeStruct((2, D_H), jnp.float32),
    ],
)

_tc_l2 = pl.pallas_call(
    _tc_l2_body,
    grid=(GRID,),
    in_specs=[_dinv_spec, _row256_spec, _full((2, D_H)), _full((1, D_H)),
              _full((1, D_H)), _full((D_H, D_OUT))],
    out_specs=_row128_spec,
    out_shape=jax.ShapeDtypeStruct((N, D_OUT), jnp.float32),
)

_tc_out = pl.pallas_call(
    _tc_out_body,
    grid=(GRID,),
    in_specs=[_dinv_spec, _vp_spec, _full((1, D_OUT))],
    out_specs=_row128_spec,
    out_shape=jax.ShapeDtypeStruct((N, D_OUT), jnp.float32),
)


def kernel(node_feat, edge_index, W1, b1, gamma, beta, W2, b2):
    del b1
    src2 = edge_index[0].reshape(NW, NCHUNK, CHUNK)
    dst2 = edge_index[1].reshape(NW, NCHUNK, CHUNK)
    zeros128 = jnp.zeros((N, D_IN), jnp.float32)

    degp = _deg_kernel(dst2, zeros128, jnp.ones((CHUNK, DW), jnp.float32))

    u1, dinv = _tc_scale(degp, node_feat)
    v1p = _agg_kernel(u1, src2, dst2, zeros128)
    h, stats = _tc_l1(dinv, v1p, W1)
    u2 = _tc_l2(dinv, h, stats, gamma.reshape(1, D_H), beta.reshape(1, D_H), W2)
    v2p = _agg_kernel(u2, src2, dst2, zeros128)
    return _tc_out(dinv, v2p, b2.reshape(1, D_OUT))
